# Initial kernel scaffold; baseline (speedup 1.0000x reference)
import jax
import jax.numpy as jnp
from jax.experimental import pallas as pl


def kernel(x, edge_index, batch, W1, b1, W2, b2, lin1_W, lin1_b, lin2_W, lin2_b, bn1_g, bn1_b, bn2_g, bn2_b, bn3_g, bn3_b, bn4_g, bn4_b, a1, a2, a3, a4):
    raise NotImplementedError("write your pallas kernel here")


# speedup vs baseline: 10.7980x; 10.7980x over previous
import jax
import jax.numpy as jnp
from jax import lax
from jax.experimental import pallas as pl
from jax.experimental.pallas import tpu as pltpu
from jax.experimental.pallas import tpu_sc as plsc

N = 100000
E = 1600000
F = 16
H = 64
B = 128

NC = 2
NS = 16
NW = NC * NS

N_PAD = 100352
NBLK = N_PAD // 1024
EW = 50176
E_PAD = EW * NW
EROWS = E_PAD // 128
ROWS_W = EW // 128
CHUNK = 8
NCH = ROWS_W // CHUNK
RPS = N_PAD // NS
ZROWS = 784
NEG = -3.0e38


def _make_sc_agg(G):
  mesh = plsc.VectorSubcoreMesh(core_axis_name="c", subcore_axis_name="s",
                                num_cores=NC, num_subcores=NS)

  def body(*refs):
    tables = refs[:G]
    src2d, dst2d, out_hbm = refs[G], refs[G + 1], refs[G + 2]
    src_v, dst_v, rows_v, zbuf, acc, sem = refs[G + 3:G + 9]

    c = lax.axis_index("c")
    s = lax.axis_index("s")
    wid = s * NC + c

    def zb(i, carry):
      zbuf[i, :] = jnp.zeros((16,), jnp.float32)
      return carry
    lax.fori_loop(0, ZROWS, zb, 0)

    for g in range(G):
      for r in range(RPS // ZROWS):
        pltpu.sync_copy(zbuf, acc.at[pl.ds(s * RPS + r * ZROWS, ZROWS)])
      plsc.subcore_barrier()

      def chunk_body(ch, carry):
        rbase = wid * ROWS_W + ch * CHUNK
        pltpu.sync_copy(src2d.at[pl.ds(rbase, CHUNK)], src_v)
        pltpu.sync_copy(dst2d.at[pl.ds(rbase, CHUNK)], dst_v)
        for j in range(CHUNK):
          pltpu.async_copy(tables[g].at[src_v.at[j]], rows_v, sem).wait()
          pltpu.sync_copy(rows_v, acc.at[dst_v.at[j]], add=True)
        return carry
      lax.fori_loop(0, NCH, chunk_body, 0)

      plsc.subcore_barrier()
      obase = (c * G + g) * N_PAD + s * RPS
      pltpu.sync_copy(acc.at[pl.ds(s * RPS, RPS)],
                      out_hbm.at[pl.ds(obase, RPS)])
      plsc.subcore_barrier()

  return pl.kernel(
      body,
      out_type=jax.ShapeDtypeStruct((NC * G * N_PAD, 16), jnp.float32),
      mesh=mesh,
      scratch_types=[
          pltpu.VMEM((CHUNK, 128), jnp.int32),
          pltpu.VMEM((CHUNK, 128), jnp.int32),
          pltpu.VMEM((128, 16), jnp.float32),
          pltpu.VMEM((ZROWS, 16), jnp.float32),
          pltpu.VMEM_SHARED((N_PAD, 16), jnp.float32),
          pltpu.SemaphoreType.DMA,
      ],
      compiler_params=pltpu.CompilerParams(use_tc_tiling_on_sc=False),
  )


import functools as _functools
_make_sc_agg = _functools.lru_cache(maxsize=None)(_make_sc_agg)


def _sc_agg1(*args):
  return _make_sc_agg(1)(*args)


def _sc_agg4(*args):
  return _make_sc_agg(4)(*args)


def _blk16(off=0):
  return pl.BlockSpec((1024, 16), lambda i, off=off: (off + i, 0))


def _blk64():
  return pl.BlockSpec((1024, 64), lambda i: (i, 0))


def _full(shape):
  return pl.BlockSpec(shape, lambda i: tuple(0 for _ in shape))


def _tca_body(p0, p1, x, dinv_out, y_out):
  dinv = lax.rsqrt(p0[...] + p1[...] + 1.0)
  dinv_out[...] = dinv
  x_rt = x[...].astype(jnp.bfloat16).astype(jnp.float32)
  y_out[...] = x_rt * dinv


def _tca(pones, x_pad):
  return pl.pallas_call(
      _tca_body,
      grid=(NBLK,),
      in_specs=[_blk16(0), _blk16(NBLK), _blk16()],
      out_specs=[_blk16(), _blk16()],
      out_shape=[jax.ShapeDtypeStruct((N_PAD, 16), jnp.float32)] * 2,
  )(pones, pones, x_pad)


def _make_conv_mm(G):
  def body(*refs):
    i = pl.program_id(0)
    zs = []
    for g in range(G):
      pa = refs[g][...]
      pb = refs[G + g][...]
      sf = refs[2 * G + g][...]
      zs.append(pa + pb + sf)
    dinv = refs[3 * G][...]
    W = refs[3 * G + 1][...]
    bvec = refs[3 * G + 2][...]
    h_out = refs[3 * G + 3]
    st_out = refs[3 * G + 4]
    acc = refs[3 * G + 5]
    z = jnp.concatenate([zg * dinv for zg in zs], axis=1)
    W_rt = W.astype(jnp.bfloat16).astype(jnp.float32)
    h = jnp.dot(z, W_rt, preferred_element_type=jnp.float32,
                precision=lax.Precision.HIGHEST) + bvec
    h_out[...] = h
    rows = lax.broadcasted_iota(jnp.int32, (1024, 1), 0) + i * 1024
    m = (rows < N).astype(jnp.float32)
    hm = h * m

    @pl.when(i == 0)
    def _():
      acc[...] = jnp.zeros_like(acc)
    acc[0:1, :] += jnp.sum(hm, axis=0, keepdims=True)
    acc[1:2, :] += jnp.sum(hm * h, axis=0, keepdims=True)

    @pl.when(i == NBLK - 1)
    def _():
      st_out[...] = acc[...]

  def run(parts, selfs, dinv, W, bvec):
    K = 16 * G
    in_specs = ([_blk16((c * G + g) * NBLK) for c in range(NC)
                 for g in range(G)]
                + [_blk16() for _ in range(G)]
                + [_blk16(), _full((K, H)), _full((1, H))])
    return pl.pallas_call(
        body,
        grid=(NBLK,),
        in_specs=in_specs,
        out_specs=[_blk64(), _full((2, H))],
        out_shape=[jax.ShapeDtypeStruct((N_PAD, H), jnp.float32),
                   jax.ShapeDtypeStruct((2, H), jnp.float32)],
        scratch_shapes=[pltpu.VMEM((2, H), jnp.float32)],
    )(*([parts] * (NC * G)), *selfs, dinv, W, bvec)

  return run


_conv_mm1 = _make_conv_mm(1)


def _aggbias_body(*refs):
  G = 4
  i = pl.program_id(0)
  zs = []
  for g in range(G):
    zs.append(refs[g][...] + refs[G + g][...] + refs[2 * G + g][...])
  dinv = refs[3 * G][...]
  bvec = refs[3 * G + 1][...]
  h_out = refs[3 * G + 2]
  st_out = refs[3 * G + 3]
  acc = refs[3 * G + 4]
  h = jnp.concatenate([zg * dinv for zg in zs], axis=1) + bvec
  h_out[...] = h
  rows = lax.broadcasted_iota(jnp.int32, (1024, 1), 0) + i * 1024
  m = (rows < N).astype(jnp.float32)
  hm = h * m

  @pl.when(i == 0)
  def _():
    acc[...] = jnp.zeros_like(acc)
  acc[0:1, :] += jnp.sum(hm, axis=0, keepdims=True)
  acc[1:2, :] += jnp.sum(hm * h, axis=0, keepdims=True)

  @pl.when(i == NBLK - 1)
  def _():
    st_out[...] = acc[...]


def _aggbias4(parts, selfs, dinv, bvec):
  G = 4
  in_specs = ([_blk16((c * G + g) * NBLK) for c in range(NC)
               for g in range(G)]
              + [_blk16() for _ in range(G)]
              + [_blk16(), _full((1, H))])
  return pl.pallas_call(
      _aggbias_body,
      grid=(NBLK,),
      in_specs=in_specs,
      out_specs=[_blk64(), _full((2, H))],
      out_shape=[jax.ShapeDtypeStruct((N_PAD, H), jnp.float32),
                 jax.ShapeDtypeStruct((2, H), jnp.float32)],
      scratch_shapes=[pltpu.VMEM((2, H), jnp.float32)],
  )(*([parts] * (NC * G)), *selfs, dinv, bvec)


def _bn_apply(h, st, g, bvec, a):
  mu = st[0:1, :] / N
  var = st[1:2, :] / N - mu * mu
  hn = (h - mu) * lax.rsqrt(var + 1e-5) * g + bvec
  return jnp.where(hn >= 0, hn, a * hn)


def _tcc_body(h_ref, st_ref, g_ref, b_ref, a_ref, dinv_ref, w2_ref,
              t0, t1, t2, t3):
  h1 = _bn_apply(h_ref[...], st_ref[...], g_ref[...], b_ref[...],
                 a_ref[0, 0])
  xw2 = jnp.dot(h1, w2_ref[...], preferred_element_type=jnp.float32)
  dinv = dinv_ref[...]
  outs = (t0, t1, t2, t3)
  for g in range(4):
    outs[g][...] = xw2[:, g * 16:(g + 1) * 16] * dinv


def _tcc(h1pre, st1, bn_g, bn_b, a, dinv, W2):
  return pl.pallas_call(
      _tcc_body,
      grid=(NBLK,),
      in_specs=[_blk64(), _full((2, H)), _full((1, H)), _full((1, H)),
                _full((1, 1)), _blk16(), _full((H, H))],
      out_specs=[_blk16()] * 4,
      out_shape=[jax.ShapeDtypeStruct((N_PAD, 16), jnp.float32)] * 4,
  )(h1pre, st1, bn_g, bn_b, a, dinv, W2)


def _tce_body(h_ref, st_ref, g_ref, b_ref, a_ref, batch_ref,
              hs_out, endidx_out, empty_out, cle, ceq, cmax, cb):
  i = pl.program_id(0)
  h = _bn_apply(h_ref[...], st_ref[...], g_ref[...], b_ref[...],
                a_ref[0, 0])
  bid = batch_ref[:, 0:1]

  io = lax.broadcasted_iota(jnp.int32, (1, B), 1)
  le = jnp.sum((bid <= io).astype(jnp.float32), axis=0, keepdims=True)
  eq = jnp.sum((bid == io).astype(jnp.float32), axis=0, keepdims=True)

  @pl.when(i == 0)
  def _():
    cle[...] = jnp.zeros_like(cle)
    ceq[...] = jnp.zeros_like(ceq)
    cmax[...] = jnp.full_like(cmax, NEG)
    cb[...] = jnp.full_like(cb, -1)
  cle[...] += le
  ceq[...] += eq

  v = jnp.where(bid == cb[0, 0], jnp.maximum(h, cmax[...]), h)
  sh = 1
  while sh < 1024:
    v_sh = jnp.concatenate(
        [jnp.full((sh, H), NEG, jnp.float32), v[:-sh, :]], axis=0)
    b_sh = jnp.concatenate(
        [jnp.full((sh, 1), -1, jnp.int32), bid[:-sh, :]], axis=0)
    v = jnp.where(b_sh == bid, jnp.maximum(v, v_sh), v)
    sh *= 2
  hs_out[...] = v
  cmax[...] = v[1023:1024, :]
  cb[...] = bid[1023:1024, :]

  @pl.when(i == NBLK - 1)
  def _():
    endidx_out[...] = jnp.maximum(cle[...] - 1.0, 0.0).astype(jnp.int32)
    empty_out[...] = (ceq[...] == 0.0).astype(jnp.float32)


def _tce(h2pre, st2, bn_g, bn_b, a, batch8):
  return pl.pallas_call(
      _tce_body,
      grid=(NBLK,),
      in_specs=[_blk64(), _full((2, H)), _full((1, H)), _full((1, H)),
                _full((1, 1)),
                pl.BlockSpec((1024, 8), lambda i: (i, 0))],
      out_specs=[_blk64(), _full((1, B)), _full((1, B))],
      out_shape=[jax.ShapeDtypeStruct((N_PAD, H), jnp.float32),
                 jax.ShapeDtypeStruct((1, B), jnp.int32),
                 jax.ShapeDtypeStruct((1, B), jnp.float32)],
      scratch_shapes=[pltpu.VMEM((1, B), jnp.float32),
                      pltpu.VMEM((1, B), jnp.float32),
                      pltpu.VMEM((1, H), jnp.float32),
                      pltpu.VMEM((1, 1), jnp.int32)],
  )(h2pre, st2, bn_g, bn_b, a, batch8)


def _gather_body(idx_ref, row_ref, out_ref):
  out_ref[...] = row_ref[...]


def _tcf(endidx, h2s3):
  grid_spec = pltpu.PrefetchScalarGridSpec(
      num_scalar_prefetch=1,
      grid=(B,),
      in_specs=[pl.BlockSpec((1, 1, H), lambda i, idx: (idx[i], 0, 0))],
      out_specs=pl.BlockSpec((1, 1, H), lambda i, idx: (i, 0, 0)),
  )
  return pl.pallas_call(
      _gather_body,
      grid_spec=grid_spec,
      out_shape=jax.ShapeDtypeStruct((B, 1, H), jnp.float32),
  )(endidx, h2s3)


def _tcg_body(p_ref, emp_ref, w1, bv1, g3, b3, a3, w2, bv2, g4, b4, a4,
              out_ref):
  p = jnp.where(emp_ref[...] > 0, -jnp.inf, p_ref[...])
  p = jnp.dot(p, w1[...], preferred_element_type=jnp.float32) + bv1[...]
  mu = jnp.mean(p, axis=0, keepdims=True)
  var = jnp.mean((p - mu) ** 2, axis=0, keepdims=True)
  p = (p - mu) * lax.rsqrt(var + 1e-5) * g3[...] + b3[...]
  p = jnp.where(p >= 0, p, a3[0, 0] * p)
  p = jnp.dot(p, w2[...], preferred_element_type=jnp.float32) + bv2[...]
  mu = jnp.mean(p, axis=0, keepdims=True)
  var = jnp.mean((p - mu) ** 2, axis=0, keepdims=True)
  p = (p - mu) * lax.rsqrt(var + 1e-5) * g4[...] + b4[...]
  out_ref[...] = jnp.where(p >= 0, p, a4[0, 0] * p)


def _tcg(gath, empty_col, lin1_W, lin1_b, bn3_g, bn3_b, a3,
         lin2_W, lin2_b, bn4_g, bn4_b, a4):
  return pl.pallas_call(
      _tcg_body,
      out_shape=jax.ShapeDtypeStruct((B, 1), jnp.float32),
  )(gath, empty_col, lin1_W, lin1_b, bn3_g, bn3_b, a3,
    lin2_W, lin2_b, bn4_g, bn4_b, a4)


def kernel(x, edge_index, batch, W1, b1, W2, b2, lin1_W, lin1_b, lin2_W,
           lin2_b, bn1_g, bn1_b, bn2_g, bn2_b, bn3_g, bn3_b, bn4_g, bn4_b,
           a1, a2, a3, a4):
  f32 = jnp.float32
  x_pad = jnp.pad(x, ((0, N_PAD - N), (0, 0)))
  src = jnp.concatenate(
      [edge_index[0], jnp.zeros((E_PAD - E,), jnp.int32)]).reshape(EROWS, 128)
  dst = jnp.concatenate(
      [edge_index[1], jnp.full((E_PAD - E,), N, jnp.int32)]).reshape(EROWS,
                                                                     128)
  batch8 = jnp.broadcast_to(
      jnp.pad(batch, (0, N_PAD - N), constant_values=B)[:, None], (N_PAD, 8))
  ones_t = jnp.ones((N_PAD, 16), f32)

  b1r = b1.reshape(1, H)
  b2r = b2.reshape(1, H)
  a1r, a2r, a3r, a4r = (a.reshape(1, 1) for a in (a1, a2, a3, a4))

  pdeg = _sc_agg1(ones_t, src, dst)
  dinv, y16 = _tca(pdeg, x_pad)

  pc1 = _sc_agg1(y16, src, dst)
  h1pre, st1 = _conv_mm1(pc1, [y16], dinv, W1, b1r)
  t0, t1, t2, t3 = _tcc(h1pre, st1, bn1_g.reshape(1, H),
                        bn1_b.reshape(1, H), a1r, dinv, W2)

  pc2 = _sc_agg4(t0, t1, t2, t3, src, dst)
  h2pre, st2 = _aggbias4(pc2, [t0, t1, t2, t3], dinv, b2r)
  h2s, endidx, empty = _tce(h2pre, st2, bn2_g.reshape(1, H),
                            bn2_b.reshape(1, H), a2r, batch8)

  gath = _tcf(endidx.reshape(B), h2s.reshape(N_PAD, 1, H)).reshape(B, H)
  out = _tcg(gath, empty.reshape(B, 1), lin1_W, lin1_b.reshape(1, H),
             bn3_g.reshape(1, H), bn3_b.reshape(1, H), a3r,
             lin2_W, lin2_b.reshape(1, 1), bn4_g.reshape(1, 1),
             bn4_b.reshape(1, 1), a4r)
  return out


# speedup vs baseline: 11.5843x; 1.0728x over previous
import jax
import jax.numpy as jnp
from jax import lax
from jax.experimental import pallas as pl
from jax.experimental.pallas import tpu as pltpu
from jax.experimental.pallas import tpu_sc as plsc

N = 100000
E = 1600000
F = 16
H = 64
B = 128

NC = 2
NS = 16
NW = NC * NS

N_PAD = 100352
NBLK = N_PAD // 1024
EW = 50176
E_PAD = EW * NW
EROWS = E_PAD // 128
ROWS_W = EW // 128
CHUNK = 8
NCH = ROWS_W // CHUNK
RPS = N_PAD // NS
ZROWS = 784
NEG = -3.0e38


def _make_sc_agg(G):
  mesh = plsc.VectorSubcoreMesh(core_axis_name="c", subcore_axis_name="s",
                                num_cores=NC, num_subcores=NS)

  def body(*refs):
    tables = refs[:G]
    src2d, dst2d, out_hbm = refs[G], refs[G + 1], refs[G + 2]
    src_v, dst_v, rows0, rows1, zbuf, acc, sem0, sem1 = refs[G + 3:G + 11]

    c = lax.axis_index("c")
    s = lax.axis_index("s")
    wid = s * NC + c

    def zb(i, carry):
      zbuf[i, :] = jnp.zeros((16,), jnp.float32)
      return carry
    lax.fori_loop(0, ZROWS, zb, 0)

    for g in range(G):
      for r in range(RPS // ZROWS):
        pltpu.sync_copy(zbuf, acc.at[pl.ds(s * RPS + r * ZROWS, ZROWS)])
      plsc.subcore_barrier()

      def chunk_body(ch, carry):
        rbase = wid * ROWS_W + ch * CHUNK
        pltpu.sync_copy(src2d.at[pl.ds(rbase, CHUNK)], src_v)
        pltpu.sync_copy(dst2d.at[pl.ds(rbase, CHUNK)], dst_v)
        bufs = (rows0, rows1)
        sems = (sem0, sem1)
        descs = [None] * CHUNK
        descs[0] = pltpu.async_copy(tables[g].at[src_v.at[0]], rows0, sem0)
        for j in range(CHUNK):
          descs[j].wait()
          if j + 1 < CHUNK:
            descs[j + 1] = pltpu.async_copy(
                tables[g].at[src_v.at[j + 1]], bufs[(j + 1) % 2],
                sems[(j + 1) % 2])
          pltpu.sync_copy(bufs[j % 2], acc.at[dst_v.at[j]], add=True)
        return carry
      lax.fori_loop(0, NCH, chunk_body, 0)

      plsc.subcore_barrier()
      obase = (c * G + g) * N_PAD + s * RPS
      pltpu.sync_copy(acc.at[pl.ds(s * RPS, RPS)],
                      out_hbm.at[pl.ds(obase, RPS)])
      plsc.subcore_barrier()

  return pl.kernel(
      body,
      out_type=jax.ShapeDtypeStruct((NC * G * N_PAD, 16), jnp.float32),
      mesh=mesh,
      scratch_types=[
          pltpu.VMEM((CHUNK, 128), jnp.int32),
          pltpu.VMEM((CHUNK, 128), jnp.int32),
          pltpu.VMEM((128, 16), jnp.float32),
          pltpu.VMEM((128, 16), jnp.float32),
          pltpu.VMEM((ZROWS, 16), jnp.float32),
          pltpu.VMEM_SHARED((N_PAD, 16), jnp.float32),
          pltpu.SemaphoreType.DMA,
          pltpu.SemaphoreType.DMA,
      ],
      compiler_params=pltpu.CompilerParams(use_tc_tiling_on_sc=False),
  )


import functools as _functools
_make_sc_agg = _functools.lru_cache(maxsize=None)(_make_sc_agg)


def _sc_agg1(*args):
  return _make_sc_agg(1)(*args)


def _sc_agg4(*args):
  return _make_sc_agg(4)(*args)


def _blk16(off=0):
  return pl.BlockSpec((1024, 16), lambda i, off=off: (off + i, 0))


def _blk64():
  return pl.BlockSpec((1024, 64), lambda i: (i, 0))


def _full(shape):
  return pl.BlockSpec(shape, lambda i: tuple(0 for _ in shape))


def _tca_body(p0, p1, x, dinv_out, y_out):
  dinv = lax.rsqrt(p0[...] + p1[...] + 1.0)
  dinv_out[...] = dinv
  x_rt = x[...].astype(jnp.bfloat16).astype(jnp.float32)
  y_out[...] = x_rt * dinv


def _tca(pones, x_pad):
  return pl.pallas_call(
      _tca_body,
      grid=(NBLK,),
      in_specs=[_blk16(0), _blk16(NBLK), _blk16()],
      out_specs=[_blk16(), _blk16()],
      out_shape=[jax.ShapeDtypeStruct((N_PAD, 16), jnp.float32)] * 2,
  )(pones, pones, x_pad)


def _make_conv_mm(G):
  def body(*refs):
    i = pl.program_id(0)
    zs = []
    for g in range(G):
      pa = refs[g][...]
      pb = refs[G + g][...]
      sf = refs[2 * G + g][...]
      zs.append(pa + pb + sf)
    dinv = refs[3 * G][...]
    W = refs[3 * G + 1][...]
    bvec = refs[3 * G + 2][...]
    h_out = refs[3 * G + 3]
    st_out = refs[3 * G + 4]
    acc = refs[3 * G + 5]
    z = jnp.concatenate([zg * dinv for zg in zs], axis=1)
    W_rt = W.astype(jnp.bfloat16).astype(jnp.float32)
    h = jnp.dot(z, W_rt, preferred_element_type=jnp.float32,
                precision=lax.Precision.HIGHEST) + bvec
    h_out[...] = h
    rows = lax.broadcasted_iota(jnp.int32, (1024, 1), 0) + i * 1024
    m = (rows < N).astype(jnp.float32)
    hm = h * m

    @pl.when(i == 0)
    def _():
      acc[...] = jnp.zeros_like(acc)
    acc[0:1, :] += jnp.sum(hm, axis=0, keepdims=True)
    acc[1:2, :] += jnp.sum(hm * h, axis=0, keepdims=True)

    @pl.when(i == NBLK - 1)
    def _():
      st_out[...] = acc[...]

  def run(parts, selfs, dinv, W, bvec):
    K = 16 * G
    in_specs = ([_blk16((c * G + g) * NBLK) for c in range(NC)
                 for g in range(G)]
                + [_blk16() for _ in range(G)]
                + [_blk16(), _full((K, H)), _full((1, H))])
    return pl.pallas_call(
        body,
        grid=(NBLK,),
        in_specs=in_specs,
        out_specs=[_blk64(), _full((2, H))],
        out_shape=[jax.ShapeDtypeStruct((N_PAD, H), jnp.float32),
                   jax.ShapeDtypeStruct((2, H), jnp.float32)],
        scratch_shapes=[pltpu.VMEM((2, H), jnp.float32)],
    )(*([parts] * (NC * G)), *selfs, dinv, W, bvec)

  return run


_conv_mm1 = _make_conv_mm(1)


def _aggbias_body(*refs):
  G = 4
  i = pl.program_id(0)
  zs = []
  for g in range(G):
    zs.append(refs[g][...] + refs[G + g][...] + refs[2 * G + g][...])
  dinv = refs[3 * G][...]
  bvec = refs[3 * G + 1][...]
  h_out = refs[3 * G + 2]
  st_out = refs[3 * G + 3]
  acc = refs[3 * G + 4]
  h = jnp.concatenate([zg * dinv for zg in zs], axis=1) + bvec
  h_out[...] = h
  rows = lax.broadcasted_iota(jnp.int32, (1024, 1), 0) + i * 1024
  m = (rows < N).astype(jnp.float32)
  hm = h * m

  @pl.when(i == 0)
  def _():
    acc[...] = jnp.zeros_like(acc)
  acc[0:1, :] += jnp.sum(hm, axis=0, keepdims=True)
  acc[1:2, :] += jnp.sum(hm * h, axis=0, keepdims=True)

  @pl.when(i == NBLK - 1)
  def _():
    st_out[...] = acc[...]


def _aggbias4(parts, selfs, dinv, bvec):
  G = 4
  in_specs = ([_blk16((c * G + g) * NBLK) for c in range(NC)
               for g in range(G)]
              + [_blk16() for _ in range(G)]
              + [_blk16(), _full((1, H))])
  return pl.pallas_call(
      _aggbias_body,
      grid=(NBLK,),
      in_specs=in_specs,
      out_specs=[_blk64(), _full((2, H))],
      out_shape=[jax.ShapeDtypeStruct((N_PAD, H), jnp.float32),
                 jax.ShapeDtypeStruct((2, H), jnp.float32)],
      scratch_shapes=[pltpu.VMEM((2, H), jnp.float32)],
  )(*([parts] * (NC * G)), *selfs, dinv, bvec)


def _bn_apply(h, st, g, bvec, a):
  mu = st[0:1, :] / N
  var = st[1:2, :] / N - mu * mu
  hn = (h - mu) * lax.rsqrt(var + 1e-5) * g + bvec
  return jnp.where(hn >= 0, hn, a * hn)


def _tcc_body(h_ref, st_ref, g_ref, b_ref, a_ref, dinv_ref, w2_ref,
              t0, t1, t2, t3):
  h1 = _bn_apply(h_ref[...], st_ref[...], g_ref[...], b_ref[...],
                 a_ref[0, 0])
  xw2 = jnp.dot(h1, w2_ref[...], preferred_element_type=jnp.float32)
  dinv = dinv_ref[...]
  outs = (t0, t1, t2, t3)
  for g in range(4):
    outs[g][...] = xw2[:, g * 16:(g + 1) * 16] * dinv


def _tcc(h1pre, st1, bn_g, bn_b, a, dinv, W2):
  return pl.pallas_call(
      _tcc_body,
      grid=(NBLK,),
      in_specs=[_blk64(), _full((2, H)), _full((1, H)), _full((1, H)),
                _full((1, 1)), _blk16(), _full((H, H))],
      out_specs=[_blk16()] * 4,
      out_shape=[jax.ShapeDtypeStruct((N_PAD, 16), jnp.float32)] * 4,
  )(h1pre, st1, bn_g, bn_b, a, dinv, W2)


def _tce_body(h_ref, st_ref, g_ref, b_ref, a_ref, batch_ref,
              hs_out, endidx_out, empty_out, cle, ceq, cmax, cb):
  i = pl.program_id(0)
  h = _bn_apply(h_ref[...], st_ref[...], g_ref[...], b_ref[...],
                a_ref[0, 0])
  bid = batch_ref[:, 0:1]

  io = lax.broadcasted_iota(jnp.int32, (1, B), 1)
  le = jnp.sum((bid <= io).astype(jnp.float32), axis=0, keepdims=True)
  eq = jnp.sum((bid == io).astype(jnp.float32), axis=0, keepdims=True)

  @pl.when(i == 0)
  def _():
    cle[...] = jnp.zeros_like(cle)
    ceq[...] = jnp.zeros_like(ceq)
    cmax[...] = jnp.full_like(cmax, NEG)
    cb[...] = jnp.full_like(cb, -1)
  cle[...] += le
  ceq[...] += eq

  v = jnp.where(bid == cb[0, 0], jnp.maximum(h, cmax[...]), h)
  sh = 1
  while sh < 1024:
    v_sh = jnp.concatenate(
        [jnp.full((sh, H), NEG, jnp.float32), v[:-sh, :]], axis=0)
    b_sh = jnp.concatenate(
        [jnp.full((sh, 1), -1, jnp.int32), bid[:-sh, :]], axis=0)
    v = jnp.where(b_sh == bid, jnp.maximum(v, v_sh), v)
    sh *= 2
  hs_out[...] = v
  cmax[...] = v[1023:1024, :]
  cb[...] = bid[1023:1024, :]

  @pl.when(i == NBLK - 1)
  def _():
    endidx_out[...] = jnp.maximum(cle[...] - 1.0, 0.0).astype(jnp.int32)
    empty_out[...] = (ceq[...] == 0.0).astype(jnp.float32)


def _tce(h2pre, st2, bn_g, bn_b, a, batch8):
  return pl.pallas_call(
      _tce_body,
      grid=(NBLK,),
      in_specs=[_blk64(), _full((2, H)), _full((1, H)), _full((1, H)),
                _full((1, 1)),
                pl.BlockSpec((1024, 8), lambda i: (i, 0))],
      out_specs=[_blk64(), _full((1, B)), _full((1, B))],
      out_shape=[jax.ShapeDtypeStruct((N_PAD, H), jnp.float32),
                 jax.ShapeDtypeStruct((1, B), jnp.int32),
                 jax.ShapeDtypeStruct((1, B), jnp.float32)],
      scratch_shapes=[pltpu.VMEM((1, B), jnp.float32),
                      pltpu.VMEM((1, B), jnp.float32),
                      pltpu.VMEM((1, H), jnp.float32),
                      pltpu.VMEM((1, 1), jnp.int32)],
  )(h2pre, st2, bn_g, bn_b, a, batch8)


def _gather_body(idx_ref, row_ref, out_ref):
  out_ref[...] = row_ref[...]


def _tcf(endidx, h2s3):
  grid_spec = pltpu.PrefetchScalarGridSpec(
      num_scalar_prefetch=1,
      grid=(B,),
      in_specs=[pl.BlockSpec((1, 1, H), lambda i, idx: (idx[i], 0, 0))],
      out_specs=pl.BlockSpec((1, 1, H), lambda i, idx: (i, 0, 0)),
  )
  return pl.pallas_call(
      _gather_body,
      grid_spec=grid_spec,
      out_shape=jax.ShapeDtypeStruct((B, 1, H), jnp.float32),
  )(endidx, h2s3)


def _tcg_body(p_ref, emp_ref, w1, bv1, g3, b3, a3, w2, bv2, g4, b4, a4,
              out_ref):
  p = jnp.where(emp_ref[...] > 0, -jnp.inf, p_ref[...])
  p = jnp.dot(p, w1[...], preferred_element_type=jnp.float32) + bv1[...]
  mu = jnp.mean(p, axis=0, keepdims=True)
  var = jnp.mean((p - mu) ** 2, axis=0, keepdims=True)
  p = (p - mu) * lax.rsqrt(var + 1e-5) * g3[...] + b3[...]
  p = jnp.where(p >= 0, p, a3[0, 0] * p)
  p = jnp.dot(p, w2[...], preferred_element_type=jnp.float32) + bv2[...]
  mu = jnp.mean(p, axis=0, keepdims=True)
  var = jnp.mean((p - mu) ** 2, axis=0, keepdims=True)
  p = (p - mu) * lax.rsqrt(var + 1e-5) * g4[...] + b4[...]
  out_ref[...] = jnp.where(p >= 0, p, a4[0, 0] * p)


def _tcg(gath, empty_col, lin1_W, lin1_b, bn3_g, bn3_b, a3,
         lin2_W, lin2_b, bn4_g, bn4_b, a4):
  return pl.pallas_call(
      _tcg_body,
      out_shape=jax.ShapeDtypeStruct((B, 1), jnp.float32),
  )(gath, empty_col, lin1_W, lin1_b, bn3_g, bn3_b, a3,
    lin2_W, lin2_b, bn4_g, bn4_b, a4)


def kernel(x, edge_index, batch, W1, b1, W2, b2, lin1_W, lin1_b, lin2_W,
           lin2_b, bn1_g, bn1_b, bn2_g, bn2_b, bn3_g, bn3_b, bn4_g, bn4_b,
           a1, a2, a3, a4):
  f32 = jnp.float32
  x_pad = jnp.pad(x, ((0, N_PAD - N), (0, 0)))
  src = jnp.concatenate(
      [edge_index[0], jnp.zeros((E_PAD - E,), jnp.int32)]).reshape(EROWS, 128)
  dst = jnp.concatenate(
      [edge_index[1], jnp.full((E_PAD - E,), N, jnp.int32)]).reshape(EROWS,
                                                                     128)
  batch8 = jnp.broadcast_to(
      jnp.pad(batch, (0, N_PAD - N), constant_values=B)[:, None], (N_PAD, 8))
  ones_t = jnp.ones((N_PAD, 16), f32)

  b1r = b1.reshape(1, H)
  b2r = b2.reshape(1, H)
  a1r, a2r, a3r, a4r = (a.reshape(1, 1) for a in (a1, a2, a3, a4))

  pdeg = _sc_agg1(ones_t, src, dst)
  dinv, y16 = _tca(pdeg, x_pad)

  pc1 = _sc_agg1(y16, src, dst)
  h1pre, st1 = _conv_mm1(pc1, [y16], dinv, W1, b1r)
  t0, t1, t2, t3 = _tcc(h1pre, st1, bn1_g.reshape(1, H),
                        bn1_b.reshape(1, H), a1r, dinv, W2)

  pc2 = _sc_agg4(t0, t1, t2, t3, src, dst)
  h2pre, st2 = _aggbias4(pc2, [t0, t1, t2, t3], dinv, b2r)
  h2s, endidx, empty = _tce(h2pre, st2, bn2_g.reshape(1, H),
                            bn2_b.reshape(1, H), a2r, batch8)

  gath = _tcf(endidx.reshape(B), h2s.reshape(N_PAD, 1, H)).reshape(B, H)
  out = _tcg(gath, empty.reshape(B, 1), lin1_W, lin1_b.reshape(1, H),
             bn3_g.reshape(1, H), bn3_b.reshape(1, H), a3r,
             lin2_W, lin2_b.reshape(1, 1), bn4_g.reshape(1, 1),
             bn4_b.reshape(1, 1), a4r)
  return out
